# baseline (speedup 1.0000x reference)
import functools
import math

import jax
import jax.numpy as jnp
from jax import lax
from jax.experimental import pallas as pl
from jax.experimental.pallas import tpu as pltpu
from jax.experimental.pallas import tpu_sc as plsc

B = 128
ENC = 50
D = 768
H = 12
HD = 64
FF = 3072
E = 8
L = 2
R = 3
V = 8192
T = B * R

BB = 16
NB = B // BB
VT = 2048
SCALE = 1.0 / math.sqrt(HD)

NC = 2
NS = 16
NW_USED = 24
ROWS_PER_W = T // NW_USED


def _head_sum_mat():
    di = lax.broadcasted_iota(jnp.int32, (D, H), 0)
    hi = lax.broadcasted_iota(jnp.int32, (D, H), 1)
    return (di // HD == hi).astype(jnp.float32)


def _head_expand_mat():
    hi = lax.broadcasted_iota(jnp.int32, (H, D), 0)
    di = lax.broadcasted_iota(jnp.int32, (H, D), 1)
    return (di // HD == hi).astype(jnp.float32)


def _r16(x):
    return x.astype(jnp.bfloat16).astype(jnp.float32)


_HI = jax.lax.Precision.HIGHEST


def _ln_rows(x, g, b):
    mu = jnp.mean(x, axis=-1, keepdims=True)
    xc = x - mu
    var = jnp.mean(xc * xc, axis=-1, keepdims=True)
    return xc * lax.rsqrt(var + 1e-5) * g + b


def _gather_x0(flat_emb, idx):
    mesh = plsc.VectorSubcoreMesh(core_axis_name="c", subcore_axis_name="s")

    @functools.partial(
        pl.kernel,
        mesh=mesh,
        out_type=jax.ShapeDtypeStruct((T, D), jnp.float32),
        scratch_types=[
            pltpu.VMEM((ROWS_PER_W,), jnp.int32),
            pltpu.VMEM((ROWS_PER_W, D), jnp.float32),
            pltpu.SemaphoreType.DMA,
        ],
    )
    def k(table_hbm, idx_hbm, out_hbm, idx_v, rows_v, sem):
        wid = lax.axis_index("s") * NC + lax.axis_index("c")

        @pl.when(wid < NW_USED)
        def _():
            base = wid * ROWS_PER_W
            pltpu.sync_copy(idx_hbm.at[pl.ds(base, ROWS_PER_W)], idx_v)
            pltpu.async_copy(table_hbm.at[idx_v], rows_v, sem).wait()
            pltpu.sync_copy(rows_v, out_hbm.at[pl.ds(base, ROWS_PER_W)])

    return k(flat_emb, idx)


def _attn_body(x_ref, enc_ref, saw_ref, sab_ref, caw_ref, cab_ref,
               lng_ref, lnb_ref, o_ref):
    M = _head_sum_mat()
    MT = _head_expand_mat()
    x3 = x_ref[...]
    xf = x3.reshape(BB * R, D)

    q = jnp.dot(xf, saw_ref[0]) + sab_ref[0:1, :]
    k = jnp.dot(xf, saw_ref[1]) + sab_ref[1:2, :]
    v = jnp.dot(xf, saw_ref[2]) + sab_ref[2:3, :]
    q3 = _r16(q.reshape(BB, R, D))
    k3 = _r16(k.reshape(BB, R, D))
    v3 = _r16(v.reshape(BB, R, D))
    outs = []
    for i in range(R):
        qi = q3[:, i, :]
        ss = [jnp.dot(k3[:, j, :] * qi, M, precision=_HI) * SCALE
              for j in range(i + 1)]
        m = ss[0]
        for s in ss[1:]:
            m = jnp.maximum(m, s)
        es = [jnp.exp(s - m) for s in ss]
        den = es[0]
        for e_ in es[1:]:
            den = den + e_
        oi = jnp.zeros((BB, D), jnp.float32)
        for j in range(i + 1):
            oi = oi + jnp.dot(_r16(es[j] / den), MT, precision=_HI) * v3[:, j, :]
        outs.append(oi)
    ao = jnp.stack(outs, axis=1).reshape(BB * R, D)
    ao = jnp.dot(ao, saw_ref[3]) + sab_ref[3:4, :]
    x1 = _ln_rows(xf + ao, lng_ref[0:1, :], lnb_ref[0:1, :])

    encf = enc_ref[...].reshape(BB * ENC, D)
    kc = _r16((jnp.dot(encf, caw_ref[1]) + cab_ref[1:2, :]).reshape(BB, ENC, D))
    vc = _r16((jnp.dot(encf, caw_ref[2]) + cab_ref[2:3, :]).reshape(BB, ENC, D))
    qc = _r16((jnp.dot(x1, caw_ref[0]) + cab_ref[0:1, :]).reshape(BB, R, D))
    outs = []
    for i in range(R):
        qi = qc[:, i, :]
        p = kc * qi[:, None, :]
        s3 = (jnp.dot(p.reshape(BB * ENC, D), M, precision=_HI)
              * SCALE).reshape(BB, ENC, H)
        m = jnp.max(s3, axis=1, keepdims=True)
        e3 = jnp.exp(s3 - m)
        a3 = _r16(e3 / jnp.sum(e3, axis=1, keepdims=True))
        A = jnp.dot(a3.reshape(BB * ENC, H), MT, precision=_HI).reshape(BB, ENC, D)
        outs.append(jnp.sum(A * vc, axis=1))
    co = jnp.stack(outs, axis=1).reshape(BB * R, D)
    co = jnp.dot(co, caw_ref[3]) + cab_ref[3:4, :]
    x2 = _ln_rows(x1 + co, lng_ref[1:2, :], lnb_ref[1:2, :])
    o_ref[...] = x2.reshape(BB, R, D)


def _attn_layer(x3, enc, saw, sab, caw, cab, lng, lnb):
    return pl.pallas_call(
        _attn_body,
        grid=(NB,),
        in_specs=[
            pl.BlockSpec((BB, R, D), lambda b: (b, 0, 0)),
            pl.BlockSpec((BB, ENC, D), lambda b: (b, 0, 0)),
            pl.BlockSpec((4, D, D), lambda b: (0, 0, 0)),
            pl.BlockSpec((4, D), lambda b: (0, 0)),
            pl.BlockSpec((4, D, D), lambda b: (0, 0, 0)),
            pl.BlockSpec((4, D), lambda b: (0, 0)),
            pl.BlockSpec((3, D), lambda b: (0, 0)),
            pl.BlockSpec((3, D), lambda b: (0, 0)),
        ],
        out_specs=pl.BlockSpec((BB, R, D), lambda b: (b, 0, 0)),
        out_shape=jax.ShapeDtypeStruct((B, R, D), jnp.float32),
    )(x3, enc, saw, sab, caw, cab, lng, lnb)


def _moe_body(xf_ref, rw_ref, rb_ref, w1_ref, b1_ref, w2_ref, b2_ref,
              lng_ref, lnb_ref, o_ref):
    e = pl.program_id(0)
    xf = xf_ref[...]

    logits = jnp.dot(xf, rw_ref[...]) + rb_ref[...]
    lm = jnp.max(logits, axis=-1, keepdims=True)
    ex = jnp.exp(logits - lm)
    probs = ex / jnp.sum(ex, axis=-1, keepdims=True)
    m1 = jnp.max(probs, axis=-1, keepdims=True)
    m2 = jnp.max(jnp.where(probs == m1, -1e30, probs), axis=-1, keepdims=True)
    e1 = jnp.exp(m1)
    e2 = jnp.exp(m2)
    w1c = e1 / (e1 + e2)
    w2c = e2 / (e1 + e2)
    comb = jnp.where(probs == m1, w1c, jnp.where(probs == m2, w2c, 0.0))
    onehot = (lax.broadcasted_iota(jnp.int32, (1, E), 1) == e).astype(jnp.float32)
    cw = jnp.sum(comb * onehot, axis=-1, keepdims=True)

    h = jnp.dot(xf, w1_ref[0]) + b1_ref[0]
    h = 0.5 * h * (1.0 + lax.erf(h * (1.0 / math.sqrt(2.0))))
    o = jnp.dot(h, w2_ref[0]) + b2_ref[0]
    contrib = cw * o

    @pl.when(e == 0)
    def _():
        o_ref[...] = contrib

    @pl.when(e > 0)
    def _():
        o_ref[...] = o_ref[...] + contrib

    @pl.when(e == E - 1)
    def _():
        o_ref[...] = _ln_rows(xf + o_ref[...], lng_ref[2:3, :], lnb_ref[2:3, :])


def _moe_layer(xf, rw, rb, w1, b1, w2, b2, lng, lnb):
    return pl.pallas_call(
        _moe_body,
        grid=(E,),
        in_specs=[
            pl.BlockSpec((T, D), lambda e: (0, 0)),
            pl.BlockSpec((D, E), lambda e: (0, 0)),
            pl.BlockSpec((1, E), lambda e: (0, 0)),
            pl.BlockSpec((1, D, FF), lambda e: (e, 0, 0)),
            pl.BlockSpec((1, 1, FF), lambda e: (e, 0, 0)),
            pl.BlockSpec((1, FF, D), lambda e: (e, 0, 0)),
            pl.BlockSpec((1, 1, D), lambda e: (e, 0, 0)),
            pl.BlockSpec((3, D), lambda e: (0, 0)),
            pl.BlockSpec((3, D), lambda e: (0, 0)),
        ],
        out_specs=pl.BlockSpec((T, D), lambda e: (0, 0)),
        out_shape=jax.ShapeDtypeStruct((T, D), jnp.float32),
    )(xf, rw, rb, w1, b1, w2, b2, lng, lnb)


def _head_body(x_ref, w_ref, b_ref, o_ref):
    o_ref[0] = jnp.dot(x_ref[0], w_ref[0]) + b_ref[0]


def _head(x_rt, out_w, out_b):
    return pl.pallas_call(
        _head_body,
        grid=(R, V // VT),
        in_specs=[
            pl.BlockSpec((1, B, D), lambda r, vt: (r, 0, 0)),
            pl.BlockSpec((1, D, VT), lambda r, vt: (r, 0, vt)),
            pl.BlockSpec((1, 1, VT), lambda r, vt: (r, 0, vt)),
        ],
        out_specs=pl.BlockSpec((1, B, VT), lambda r, vt: (r, 0, vt)),
        out_shape=jax.ShapeDtypeStruct((R, B, V), jnp.float32),
    )(x_rt, out_w, out_b)


def kernel(emb, sa_w, sa_b, ca_w, ca_b, router_w, router_b, e_w1, e_b1,
           e_w2, e_b2, ln_g, ln_b, out_w, out_b, encoder_output, target_ids):
    flat_emb = emb.reshape(R * V, D)
    idx = (jnp.arange(R, dtype=jnp.int32)[None, :] * V
           + target_ids.astype(jnp.int32)).reshape(-1)
    xf = _gather_x0(flat_emb, idx)
    x3 = xf.reshape(B, R, D)
    for l in range(L):
        x3 = _attn_layer(x3, encoder_output, sa_w[l], sa_b[l],
                         ca_w[l], ca_b[l], ln_g[l], ln_b[l])
        xf = _moe_layer(x3.reshape(T, D), router_w[l], router_b[l].reshape(1, E),
                        e_w1[l], e_b1[l].reshape(E, 1, FF),
                        e_w2[l], e_b2[l].reshape(E, 1, D), ln_g[l], ln_b[l])
        x3 = xf.reshape(B, R, D)
    out = _head(x3.transpose(1, 0, 2), out_w, out_b.reshape(R, 1, V))
    return out.transpose(1, 0, 2)
